# Initial kernel scaffold; baseline (speedup 1.0000x reference)
import jax
import jax.numpy as jnp
from jax.experimental import pallas as pl


def kernel(x, internal_edge_index, all_edge_index, params):
    raise NotImplementedError("write your pallas kernel here")


# speedup vs baseline: 5.9274x; 5.9274x over previous
import functools

import jax
import jax.numpy as jnp
from jax import lax
from jax.experimental import pallas as pl
from jax.experimental.pallas import tpu as pltpu
from jax.experimental.pallas import tpu_sc as plsc

N = 10000
D = 128
E = 320000
NCLS = 10
GRP = 1000

NC, NS = 2, 16
NW = NC * NS
CHUNK = 128
EPW = 10240
NCHUNK = EPW // CHUNK
EPAD = EPW * NW
DUMMY = N
NP = 10112
RPT = NP // NS

_f32 = jnp.float32


def _mesh():
    return plsc.VectorSubcoreMesh(
        core_axis_name="c", subcore_axis_name="s", num_cores=NC, num_subcores=NS
    )


@functools.partial(
    pl.kernel,
    out_type=jax.ShapeDtypeStruct((2, NC, NP, D), _f32),
    mesh=_mesh(),
    scratch_types=[
        pltpu.VMEM((NCHUNK, CHUNK), jnp.int32),
        pltpu.VMEM((CHUNK, D), _f32),
        pltpu.VMEM_SHARED((NP, D), _f32),
    ],
)
def _sc_deg(ci_hbm, ca_hbm, ones_hbm, zeros_hbm, out_hbm, col_v, ones_v, acc):
    c = lax.axis_index("c")
    s = lax.axis_index("s")
    pltpu.sync_copy(ones_hbm, ones_v)
    for k, src in ((0, ci_hbm), (1, ca_hbm)):
        pltpu.sync_copy(zeros_hbm, acc.at[pl.ds(s * RPT, RPT)])
        pltpu.sync_copy(src.at[c, s], col_v)
        plsc.subcore_barrier()

        def body(g, _):
            pltpu.sync_copy(ones_v, acc.at[col_v.at[g]], add=True)
            return 0

        lax.fori_loop(0, NCHUNK, body, 0)
        plsc.subcore_barrier()
        pltpu.sync_copy(
            acc.at[pl.ds(s * RPT, RPT)], out_hbm.at[k, c, pl.ds(s * RPT, RPT)]
        )
        plsc.subcore_barrier()


@functools.partial(
    pl.kernel,
    out_type=jax.ShapeDtypeStruct((NC, NP, D), _f32),
    mesh=_mesh(),
    scratch_types=[
        pltpu.VMEM((NCHUNK, CHUNK), jnp.int32),
        pltpu.VMEM((NCHUNK, CHUNK), jnp.int32),
        pltpu.VMEM((CHUNK, D), _f32),
        pltpu.VMEM_SHARED((NP, D), _f32),
        pltpu.SemaphoreType.DMA,
    ],
)
def _sc_agg(xs_hbm, row_hbm, col_hbm, zeros_hbm, out_hbm,
            row_v, col_v, buf0, acc, sem0):
    c = lax.axis_index("c")
    s = lax.axis_index("s")
    pltpu.sync_copy(row_hbm.at[c, s], row_v)
    pltpu.sync_copy(col_hbm.at[c, s], col_v)
    pltpu.sync_copy(zeros_hbm, acc.at[pl.ds(s * RPT, RPT)])
    plsc.subcore_barrier()

    def body(g, _):
        pltpu.async_copy(xs_hbm.at[row_v.at[g]], buf0, sem0).wait()
        pltpu.sync_copy(buf0, acc.at[col_v.at[g]], add=True)
        return 0

    lax.fori_loop(0, NCHUNK, body, 0)
    plsc.subcore_barrier()
    pltpu.sync_copy(
        acc.at[pl.ds(s * RPT, RPT)], out_hbm.at[c, pl.ds(s * RPT, RPT)]
    )


def _mmT(a, w):
    return lax.dot_general(a, w, (((1,), (1,)), ((), ())),
                           preferred_element_type=_f32)


def _bn_act(y, g, b, act):
    mu = jnp.sum(y, axis=0, keepdims=True) * (1.0 / N)
    yc = y - mu
    var = (jnp.sum(yc * yc, axis=0, keepdims=True) - (NP - N) * mu * mu) * (1.0 / N)
    return act(yc * lax.rsqrt(var + 1e-5) * g + b)


_relu = lambda v: jnp.maximum(v, 0.0)
_leaky = lambda v: jnp.where(v >= 0, v, 0.1 * v)


def _tc(body, *outs):
    return pl.pallas_call(
        body, out_shape=[jax.ShapeDtypeStruct(s, d) for s, d in outs]
    )


def _prep_body(x_ref, degp_ref, wit_ref, bit_ref, w1_ref, b1_ref,
               xs_ref, disi_ref, disa_ref):
    mask = (lax.broadcasted_iota(jnp.int32, (NP, 1), 0) < N).astype(_f32)
    cnt_i = degp_ref[0, 0] + degp_ref[0, 1]
    cnt_a = degp_ref[1, 0] + degp_ref[1, 1]
    dis_i = lax.rsqrt(1.0 + cnt_i) * mask
    dis_a = lax.rsqrt(1.0 + cnt_a) * mask
    xi = _mmT(x_ref[...], wit_ref[...]) + bit_ref[...]
    xs_ref[...] = dis_i * (_mmT(xi, w1_ref[...]) + b1_ref[...])
    disi_ref[...] = dis_i
    disa_ref[...] = dis_a


def _make_mid_body(act):
    def body(yp_ref, xs_ref, disc_ref, g_ref, bt_ref, w_ref, b_ref, disn_ref,
             out_ref):
        y = disc_ref[...] * (yp_ref[0] + yp_ref[1] + xs_ref[...])
        h = _bn_act(y, g_ref[...], bt_ref[...], act)
        out_ref[...] = disn_ref[...] * (_mmT(h, w_ref[...]) + b_ref[...])
    return body


def _merge_body(yp_ref, xs_ref, disi_ref, g_ref, bt_ref, x_ref, wm1_ref,
                wm2_ref, bm_ref, wg1_ref, gb1_ref, disa_ref, out_ref):
    y = disi_ref[...] * (yp_ref[0] + yp_ref[1] + xs_ref[...])
    xi3 = _bn_act(y, g_ref[...], bt_ref[...], _relu)
    xc = _mmT(x_ref[...], wm1_ref[...]) + _mmT(xi3, wm2_ref[...]) + bm_ref[...]
    out_ref[...] = disa_ref[...] * (_mmT(xc, wg1_ref[...]) + gb1_ref[...])


def _final_body(yp_ref, xs_ref, disa_ref, g_ref, bt_ref, pw_ref, wg_ref,
                bg_ref, out_ref, ypred_ref):
    y = disa_ref[...] * (yp_ref[0] + yp_ref[1] + xs_ref[...])
    xg = _bn_act(y, g_ref[...], bt_ref[...], _leaky)
    xg = lax.slice(xg, (0, 0), (N, D)).reshape(N // GRP, GRP, D)
    pw = pw_ref[0, 0]
    xg = jnp.clip(xg, 0.0, 100.0) ** pw
    xg = jnp.sum(xg, axis=1) * (1.0 / GRP)
    xg = jnp.clip(xg, 0.0, 100.0) ** (1.0 / pw)
    o = _mmT(xg, wg_ref[...]) + bg_ref[...]
    out_ref[...] = o
    iota = lax.broadcasted_iota(jnp.int32, (N // GRP, NCLS), 1)
    mx = jnp.max(o, axis=1, keepdims=True)
    ypred_ref[...] = jnp.min(jnp.where(o == mx, iota, NCLS), axis=1,
                             keepdims=True)


_tc_prep = _tc(_prep_body, ((NP, D), _f32), ((NP, 1), _f32), ((NP, 1), _f32))
_tc_mid_relu = _tc(_make_mid_body(_relu), ((NP, D), _f32))
_tc_mid_leaky = _tc(_make_mid_body(_leaky), ((NP, D), _f32))
_tc_merge = _tc(_merge_body, ((NP, D), _f32))
_tc_final = _tc(_final_body, ((N // GRP, NCLS), _f32), ((N // GRP, 1), jnp.int32))


def _prep_edges(ei):
    pad = jnp.full((EPAD - E,), DUMMY, jnp.int32)
    r = jnp.concatenate([ei[0], pad]).reshape(NC, NS, NCHUNK, CHUNK)
    c = jnp.concatenate([ei[1], pad]).reshape(NC, NS, NCHUNK, CHUNK)
    return r, c


def kernel(x, internal_edge_index, all_edge_index, params):
    p = params
    x_pad = jnp.pad(x, ((0, NP - N), (0, 0)))
    ri, ci = _prep_edges(internal_edge_index)
    ra, ca = _prep_edges(all_edge_index)
    zeros_d = jnp.zeros((RPT, D), _f32)
    ones_d = jnp.ones((CHUNK, D), _f32)
    r2 = lambda v: v.reshape(1, -1)

    degp = _sc_deg(ci, ca, ones_d, zeros_d)[:, :, :, 0:1]
    xs, dis_i, dis_a = _tc_prep(x_pad, degp, p['W_it'], r2(p['b_it']),
                                p['iW1'], r2(p['ib1']))
    yp = _sc_agg(xs, ri, ci, zeros_d)
    xs = _tc_mid_relu(yp, xs, dis_i, r2(p['ig1']), r2(p['ibt1']),
                      p['iW2'], r2(p['ib2']), dis_i)[0]
    yp = _sc_agg(xs, ri, ci, zeros_d)
    xs = _tc_mid_relu(yp, xs, dis_i, r2(p['ig2']), r2(p['ibt2']),
                      p['iW3'], r2(p['ib3']), dis_i)[0]
    yp = _sc_agg(xs, ri, ci, zeros_d)
    xs = _tc_merge(yp, xs, dis_i, r2(p['ig3']), r2(p['ibt3']), x_pad,
                   p['W_m'][:, :D], p['W_m'][:, D:], r2(p['b_m']),
                   p['gW1'], r2(p['gb1']), dis_a)[0]
    yp = _sc_agg(xs, ra, ca, zeros_d)
    xs = _tc_mid_leaky(yp, xs, dis_a, r2(p['gg1']), r2(p['gbt1']),
                       p['gW2'], r2(p['gb2']), dis_a)[0]
    yp = _sc_agg(xs, ra, ca, zeros_d)
    xs = _tc_mid_leaky(yp, xs, dis_a, r2(p['gg2']), r2(p['gbt2']),
                       p['gW3'], r2(p['gb3']), dis_a)[0]
    yp = _sc_agg(xs, ra, ca, zeros_d)
    out, ypred = _tc_final(yp, xs, dis_a, r2(p['gg3']), r2(p['gbt3']),
                           p['p_pow'].reshape(1, 1), p['Wg'], r2(p['bg']))
    return out, ypred.reshape(-1)


# speedup vs baseline: 6.3974x; 1.0793x over previous
import functools

import jax
import jax.numpy as jnp
from jax import lax
from jax.experimental import pallas as pl
from jax.experimental.pallas import tpu as pltpu
from jax.experimental.pallas import tpu_sc as plsc

N = 10000
D = 128
E = 320000
NCLS = 10
GRP = 1000

NC, NS = 2, 16
NW = NC * NS
CHUNK = 128
EPW = 10240
NCHUNK = EPW // CHUNK
EPAD = EPW * NW
DUMMY = N
NP = 10112
RPT = NP // NS

_f32 = jnp.float32


def _mesh():
    return plsc.VectorSubcoreMesh(
        core_axis_name="c", subcore_axis_name="s", num_cores=NC, num_subcores=NS
    )


@functools.partial(
    pl.kernel,
    out_type=jax.ShapeDtypeStruct((2, NC, NP, D), _f32),
    mesh=_mesh(),
    scratch_types=[
        pltpu.VMEM((NCHUNK, CHUNK), jnp.int32),
        pltpu.VMEM((CHUNK, D), _f32),
        pltpu.VMEM_SHARED((NP, D), _f32),
    ],
)
def _sc_deg(ci_hbm, ca_hbm, ones_hbm, zeros_hbm, out_hbm, col_v, ones_v, acc):
    c = lax.axis_index("c")
    s = lax.axis_index("s")
    pltpu.sync_copy(ones_hbm, ones_v)
    for k, src in ((0, ci_hbm), (1, ca_hbm)):
        pltpu.sync_copy(zeros_hbm, acc.at[pl.ds(s * RPT, RPT)])
        pltpu.sync_copy(src.at[c, s], col_v)
        plsc.subcore_barrier()

        def body(g, _):
            pltpu.sync_copy(ones_v, acc.at[col_v.at[g]], add=True)
            return 0

        lax.fori_loop(0, NCHUNK, body, 0)
        plsc.subcore_barrier()
        pltpu.sync_copy(
            acc.at[pl.ds(s * RPT, RPT)], out_hbm.at[k, c, pl.ds(s * RPT, RPT)]
        )
        plsc.subcore_barrier()


@functools.partial(
    pl.kernel,
    out_type=jax.ShapeDtypeStruct((NC, NP, D), _f32),
    mesh=_mesh(),
    scratch_types=[
        pltpu.VMEM((NCHUNK, CHUNK), jnp.int32),
        pltpu.VMEM((2, CHUNK), jnp.int32),
        pltpu.VMEM((CHUNK, D), _f32),
        pltpu.VMEM((CHUNK, D), _f32),
        pltpu.VMEM_SHARED((NP, D), _f32),
        pltpu.SemaphoreType.DMA,
        pltpu.SemaphoreType.DMA,
        pltpu.SemaphoreType.DMA,
        pltpu.SemaphoreType.DMA,
        pltpu.SemaphoreType.DMA,
        pltpu.SemaphoreType.DMA,
    ],
)
def _sc_agg(xs_hbm, row_hbm, col_hbm, zeros_hbm, out_hbm,
            row_v, col_r, buf0, buf1, acc, gs0, gs1, ss0, ss1, is0, is1):
    c = lax.axis_index("c")
    s = lax.axis_index("s")
    pltpu.sync_copy(row_hbm.at[c, s], row_v)
    pltpu.sync_copy(zeros_hbm, acc.at[pl.ds(s * RPT, RPT)])
    plsc.subcore_barrier()

    bufs = (buf0, buf1)
    gsems = (gs0, gs1)
    ssems = (ss0, ss1)
    isems = (is0, is1)

    def issue_gather(g, b):
        pltpu.async_copy(xs_hbm.at[row_v.at[g]], bufs[b], gsems[b])

    def issue_col(g, b):
        pltpu.async_copy(col_hbm.at[c, s, g], col_r.at[b], isems[b])

    def issue_scat(g, b):
        pltpu.async_copy(bufs[b], acc.at[col_r.at[b]], ssems[b], add=True)

    def drain_big(sem, b):
        pltpu.make_async_copy(zeros_hbm.at[pl.ds(0, CHUNK)], bufs[b],
                              sem).wait()

    def drain_col(b):
        pltpu.make_async_copy(col_hbm.at[c, s, 0], col_r.at[b],
                              isems[b]).wait()

    issue_gather(0, 0)
    issue_col(0, 0)

    def body(i, _):
        for par in range(2):
            g = 2 * i + par
            nxt = par ^ 1
            drain_big(gsems[par], par)

            if par == 1:
                drain_big(ssems[0], 0)
            else:
                @pl.when(i > 0)
                def _():
                    drain_big(ssems[1], 1)

            if par == 0:
                issue_gather(g + 1, nxt)
                issue_col(g + 1, nxt)
            else:
                @pl.when(i < NCHUNK // 2 - 1)
                def _():
                    issue_gather(g + 1, nxt)
                    issue_col(g + 1, nxt)

            drain_col(par)
            issue_scat(g, par)
        return 0

    lax.fori_loop(0, NCHUNK // 2, body, 0)
    drain_big(ssems[1], 1)
    plsc.subcore_barrier()
    pltpu.sync_copy(
        acc.at[pl.ds(s * RPT, RPT)], out_hbm.at[c, pl.ds(s * RPT, RPT)]
    )


def _mmT(a, w):
    return lax.dot_general(a, w, (((1,), (1,)), ((), ())),
                           preferred_element_type=_f32)


def _bn_act(y, g, b, act):
    mu = jnp.sum(y, axis=0, keepdims=True) * (1.0 / N)
    yc = y - mu
    var = (jnp.sum(yc * yc, axis=0, keepdims=True) - (NP - N) * mu * mu) * (1.0 / N)
    return act(yc * lax.rsqrt(var + 1e-5) * g + b)


_relu = lambda v: jnp.maximum(v, 0.0)
_leaky = lambda v: jnp.where(v >= 0, v, 0.1 * v)


def _tc(body, *outs):
    return pl.pallas_call(
        body, out_shape=[jax.ShapeDtypeStruct(s, d) for s, d in outs]
    )


def _prep_body(x_ref, degp_ref, wit_ref, bit_ref, w1_ref, b1_ref,
               xs_ref, disi_ref, disa_ref):
    mask = (lax.broadcasted_iota(jnp.int32, (NP, 1), 0) < N).astype(_f32)
    cnt_i = degp_ref[0, 0] + degp_ref[0, 1]
    cnt_a = degp_ref[1, 0] + degp_ref[1, 1]
    dis_i = lax.rsqrt(1.0 + cnt_i) * mask
    dis_a = lax.rsqrt(1.0 + cnt_a) * mask
    xi = _mmT(x_ref[...], wit_ref[...]) + bit_ref[...]
    xs_ref[...] = dis_i * (_mmT(xi, w1_ref[...]) + b1_ref[...])
    disi_ref[...] = dis_i
    disa_ref[...] = dis_a


def _make_mid_body(act):
    def body(yp_ref, xs_ref, disc_ref, g_ref, bt_ref, w_ref, b_ref, disn_ref,
             out_ref):
        y = disc_ref[...] * (yp_ref[0] + yp_ref[1] + xs_ref[...])
        h = _bn_act(y, g_ref[...], bt_ref[...], act)
        out_ref[...] = disn_ref[...] * (_mmT(h, w_ref[...]) + b_ref[...])
    return body


def _merge_body(yp_ref, xs_ref, disi_ref, g_ref, bt_ref, x_ref, wm1_ref,
                wm2_ref, bm_ref, wg1_ref, gb1_ref, disa_ref, out_ref):
    y = disi_ref[...] * (yp_ref[0] + yp_ref[1] + xs_ref[...])
    xi3 = _bn_act(y, g_ref[...], bt_ref[...], _relu)
    xc = _mmT(x_ref[...], wm1_ref[...]) + _mmT(xi3, wm2_ref[...]) + bm_ref[...]
    out_ref[...] = disa_ref[...] * (_mmT(xc, wg1_ref[...]) + gb1_ref[...])


def _final_body(yp_ref, xs_ref, disa_ref, g_ref, bt_ref, pw_ref, wg_ref,
                bg_ref, out_ref, ypred_ref):
    y = disa_ref[...] * (yp_ref[0] + yp_ref[1] + xs_ref[...])
    xg = _bn_act(y, g_ref[...], bt_ref[...], _leaky)
    xg = lax.slice(xg, (0, 0), (N, D)).reshape(N // GRP, GRP, D)
    pw = pw_ref[0, 0]
    xg = jnp.clip(xg, 0.0, 100.0) ** pw
    xg = jnp.sum(xg, axis=1) * (1.0 / GRP)
    xg = jnp.clip(xg, 0.0, 100.0) ** (1.0 / pw)
    o = _mmT(xg, wg_ref[...]) + bg_ref[...]
    out_ref[...] = o
    iota = lax.broadcasted_iota(jnp.int32, (N // GRP, NCLS), 1)
    mx = jnp.max(o, axis=1, keepdims=True)
    ypred_ref[...] = jnp.min(jnp.where(o == mx, iota, NCLS), axis=1,
                             keepdims=True)


_tc_prep = _tc(_prep_body, ((NP, D), _f32), ((NP, 1), _f32), ((NP, 1), _f32))
_tc_mid_relu = _tc(_make_mid_body(_relu), ((NP, D), _f32))
_tc_mid_leaky = _tc(_make_mid_body(_leaky), ((NP, D), _f32))
_tc_merge = _tc(_merge_body, ((NP, D), _f32))
_tc_final = _tc(_final_body, ((N // GRP, NCLS), _f32), ((N // GRP, 1), jnp.int32))


def _prep_edges(ei):
    pad = jnp.full((EPAD - E,), DUMMY, jnp.int32)
    r = jnp.concatenate([ei[0], pad]).reshape(NC, NS, NCHUNK, CHUNK)
    c = jnp.concatenate([ei[1], pad]).reshape(NC, NS, NCHUNK, CHUNK)
    return r, c


def kernel(x, internal_edge_index, all_edge_index, params):
    p = params
    x_pad = jnp.pad(x, ((0, NP - N), (0, 0)))
    ri, ci = _prep_edges(internal_edge_index)
    ra, ca = _prep_edges(all_edge_index)
    zeros_d = jnp.zeros((RPT, D), _f32)
    ones_d = jnp.ones((CHUNK, D), _f32)
    r2 = lambda v: v.reshape(1, -1)

    degp = _sc_deg(ci, ca, ones_d, zeros_d)[:, :, :, 0:1]
    xs, dis_i, dis_a = _tc_prep(x_pad, degp, p['W_it'], r2(p['b_it']),
                                p['iW1'], r2(p['ib1']))
    yp = _sc_agg(xs, ri, ci, zeros_d)
    xs = _tc_mid_relu(yp, xs, dis_i, r2(p['ig1']), r2(p['ibt1']),
                      p['iW2'], r2(p['ib2']), dis_i)[0]
    yp = _sc_agg(xs, ri, ci, zeros_d)
    xs = _tc_mid_relu(yp, xs, dis_i, r2(p['ig2']), r2(p['ibt2']),
                      p['iW3'], r2(p['ib3']), dis_i)[0]
    yp = _sc_agg(xs, ri, ci, zeros_d)
    xs = _tc_merge(yp, xs, dis_i, r2(p['ig3']), r2(p['ibt3']), x_pad,
                   p['W_m'][:, :D], p['W_m'][:, D:], r2(p['b_m']),
                   p['gW1'], r2(p['gb1']), dis_a)[0]
    yp = _sc_agg(xs, ra, ca, zeros_d)
    xs = _tc_mid_leaky(yp, xs, dis_a, r2(p['gg1']), r2(p['gbt1']),
                       p['gW2'], r2(p['gb2']), dis_a)[0]
    yp = _sc_agg(xs, ra, ca, zeros_d)
    xs = _tc_mid_leaky(yp, xs, dis_a, r2(p['gg2']), r2(p['gbt2']),
                       p['gW3'], r2(p['gb3']), dis_a)[0]
    yp = _sc_agg(xs, ra, ca, zeros_d)
    out, ypred = _tc_final(yp, xs, dis_a, r2(p['gg3']), r2(p['gbt3']),
                           p['p_pow'].reshape(1, 1), p['Wg'], r2(p['bg']))
    return out, ypred.reshape(-1)


# speedup vs baseline: 7.8439x; 1.2261x over previous
import functools

import jax
import jax.numpy as jnp
from jax import lax
from jax.experimental import pallas as pl
from jax.experimental.pallas import tpu as pltpu
from jax.experimental.pallas import tpu_sc as plsc

N = 10000
D = 128
E = 320000
NCLS = 10
GRP = 1000

NC, NS = 2, 16
NW = NC * NS
CHUNK = 128
EPW = 10240
NCHUNK = EPW // CHUNK
EPAD = EPW * NW
DUMMY = N
NP = 10112
RPT = NP // NS

_f32 = jnp.float32


def _mesh():
    return plsc.VectorSubcoreMesh(
        core_axis_name="c", subcore_axis_name="s", num_cores=NC, num_subcores=NS
    )


@functools.partial(
    pl.kernel,
    out_type=jax.ShapeDtypeStruct((2, NC, NP, D), _f32),
    mesh=_mesh(),
    scratch_types=[
        pltpu.VMEM((NCHUNK, CHUNK), jnp.int32),
        pltpu.VMEM((CHUNK, D), _f32),
        pltpu.VMEM_SHARED((NP, D), _f32),
    ],
)
def _sc_deg(ci_hbm, ca_hbm, ones_hbm, zeros_hbm, out_hbm, col_v, ones_v, acc):
    c = lax.axis_index("c")
    s = lax.axis_index("s")
    base = (c * NS + s) * NCHUNK
    pltpu.sync_copy(ones_hbm, ones_v)
    for k, src in ((0, ci_hbm), (1, ca_hbm)):
        pltpu.sync_copy(zeros_hbm, acc.at[pl.ds(s * RPT, RPT)])
        pltpu.sync_copy(src.at[pl.ds(base, NCHUNK)], col_v)
        plsc.subcore_barrier()

        def body(g, _):
            pltpu.sync_copy(ones_v, acc.at[col_v.at[g]], add=True)
            return 0

        lax.fori_loop(0, NCHUNK, body, 0)
        plsc.subcore_barrier()
        pltpu.sync_copy(
            acc.at[pl.ds(s * RPT, RPT)], out_hbm.at[k, c, pl.ds(s * RPT, RPT)]
        )
        plsc.subcore_barrier()


HALF_TRIPS = (64, 16)
T0C, T1C = 2 * HALF_TRIPS[0], 2 * HALF_TRIPS[1]
NROWS = EPAD // CHUNK
assert NS * (T0C + T1C) == NROWS


@functools.partial(
    pl.kernel,
    out_type=jax.ShapeDtypeStruct((NC, NP, D), _f32),
    mesh=_mesh(),
    scratch_types=[
        pltpu.VMEM((2, CHUNK), jnp.int32),
        pltpu.VMEM((2, CHUNK), jnp.int32),
        pltpu.VMEM((CHUNK, D), _f32),
        pltpu.VMEM((CHUNK, D), _f32),
        pltpu.VMEM_SHARED((NP, D), _f32),
        pltpu.SemaphoreType.DMA,
        pltpu.SemaphoreType.DMA,
        pltpu.SemaphoreType.DMA,
        pltpu.SemaphoreType.DMA,
        pltpu.SemaphoreType.DMA,
        pltpu.SemaphoreType.DMA,
        pltpu.SemaphoreType.DMA,
        pltpu.SemaphoreType.DMA,
    ],
)
def _sc_agg(xs_hbm, row_hbm, col_hbm, zeros_hbm, out_hbm,
            row_r, col_r, buf0, buf1, acc,
            gs0, gs1, ss0, ss1, rs0, rs1, cs0, cs1):
    c = lax.axis_index("c")
    s = lax.axis_index("s")
    nhalf = jnp.where(c == 0, HALF_TRIPS[0], HALF_TRIPS[1])
    base = jnp.where(c == 0, s * T0C, NS * T0C + s * T1C)
    pltpu.sync_copy(zeros_hbm, acc.at[pl.ds(s * RPT, RPT)])
    plsc.subcore_barrier()

    bufs = (buf0, buf1)
    gsems = (gs0, gs1)
    ssems = (ss0, ss1)
    rsems = (rs0, rs1)
    csems = (cs0, cs1)

    def issue_row(g, b):
        pltpu.async_copy(row_hbm.at[base + g], row_r.at[b], rsems[b])

    def issue_col(g, b):
        pltpu.async_copy(col_hbm.at[base + g], col_r.at[b], csems[b])

    def issue_gather(b):
        pltpu.async_copy(xs_hbm.at[row_r.at[b]], bufs[b], gsems[b])

    def issue_scat(b):
        pltpu.async_copy(bufs[b], acc.at[col_r.at[b]], ssems[b], add=True)

    def drain_big(sem, b):
        pltpu.make_async_copy(zeros_hbm.at[pl.ds(0, CHUNK)], bufs[b],
                              sem).wait()

    def drain_idx(sem, r, b):
        pltpu.make_async_copy(row_hbm.at[0], r.at[b], sem).wait()

    @pl.when(nhalf > 0)
    def _():
        issue_row(0, 0)
        issue_row(1, 1)
        issue_col(0, 0)
        drain_idx(rsems[0], row_r, 0)
        issue_gather(0)

    def body(i, _):
        for par in range(2):
            g = 2 * i + par
            nxt = par ^ 1
            drain_big(gsems[par], par)

            if par == 1:
                drain_big(ssems[0], 0)
            else:
                @pl.when(i > 0)
                def _():
                    drain_big(ssems[1], 1)

            @pl.when(i < nhalf - 1)
            def _():
                issue_row(g + 2, par)

            if par == 0:
                drain_idx(rsems[1], row_r, 1)
                issue_gather(1)
                issue_col(g + 1, 1)
            else:
                @pl.when(i < nhalf - 1)
                def _():
                    drain_idx(rsems[0], row_r, 0)
                    issue_gather(0)
                    issue_col(g + 1, 0)

            drain_idx(csems[par], col_r, par)
            issue_scat(par)
        return 0

    lax.fori_loop(0, nhalf, body, 0)

    @pl.when(nhalf > 0)
    def _():
        drain_big(ssems[1], 1)
    plsc.subcore_barrier()
    pltpu.sync_copy(
        acc.at[pl.ds(s * RPT, RPT)], out_hbm.at[c, pl.ds(s * RPT, RPT)]
    )


def _mmT(a, w):
    return lax.dot_general(a, w, (((1,), (1,)), ((), ())),
                           preferred_element_type=_f32)


def _bn_act(y, g, b, act):
    mu = jnp.sum(y, axis=0, keepdims=True) * (1.0 / N)
    yc = y - mu
    var = (jnp.sum(yc * yc, axis=0, keepdims=True) - (NP - N) * mu * mu) * (1.0 / N)
    return act(yc * lax.rsqrt(var + 1e-5) * g + b)


_relu = lambda v: jnp.maximum(v, 0.0)
_leaky = lambda v: jnp.where(v >= 0, v, 0.1 * v)


def _tc(body, *outs):
    return pl.pallas_call(
        body, out_shape=[jax.ShapeDtypeStruct(s, d) for s, d in outs]
    )


def _prep_body(x_ref, degp_ref, wit_ref, bit_ref, w1_ref, b1_ref,
               xs_ref, disi_ref, disa_ref):
    mask = (lax.broadcasted_iota(jnp.int32, (NP, 1), 0) < N).astype(_f32)
    cnt_i = degp_ref[0, 0] + degp_ref[0, 1]
    cnt_a = degp_ref[1, 0] + degp_ref[1, 1]
    dis_i = lax.rsqrt(1.0 + cnt_i) * mask
    dis_a = lax.rsqrt(1.0 + cnt_a) * mask
    xi = _mmT(x_ref[...], wit_ref[...]) + bit_ref[...]
    xs_ref[...] = dis_i * (_mmT(xi, w1_ref[...]) + b1_ref[...])
    disi_ref[...] = dis_i
    disa_ref[...] = dis_a


def _make_mid_body(act):
    def body(yp_ref, xs_ref, disc_ref, g_ref, bt_ref, w_ref, b_ref, disn_ref,
             out_ref):
        y = disc_ref[...] * (yp_ref[0] + yp_ref[1] + xs_ref[...])
        h = _bn_act(y, g_ref[...], bt_ref[...], act)
        out_ref[...] = disn_ref[...] * (_mmT(h, w_ref[...]) + b_ref[...])
    return body


def _merge_body(yp_ref, xs_ref, disi_ref, g_ref, bt_ref, x_ref, wm1_ref,
                wm2_ref, bm_ref, wg1_ref, gb1_ref, disa_ref, out_ref):
    y = disi_ref[...] * (yp_ref[0] + yp_ref[1] + xs_ref[...])
    xi3 = _bn_act(y, g_ref[...], bt_ref[...], _relu)
    xc = _mmT(x_ref[...], wm1_ref[...]) + _mmT(xi3, wm2_ref[...]) + bm_ref[...]
    out_ref[...] = disa_ref[...] * (_mmT(xc, wg1_ref[...]) + gb1_ref[...])


def _final_body(yp_ref, xs_ref, disa_ref, g_ref, bt_ref, pw_ref, wg_ref,
                bg_ref, out_ref, ypred_ref):
    y = disa_ref[...] * (yp_ref[0] + yp_ref[1] + xs_ref[...])
    xg = _bn_act(y, g_ref[...], bt_ref[...], _leaky)
    xg = lax.slice(xg, (0, 0), (N, D)).reshape(N // GRP, GRP, D)
    pw = pw_ref[0, 0]
    xg = jnp.clip(xg, 0.0, 100.0) ** pw
    xg = jnp.sum(xg, axis=1) * (1.0 / GRP)
    xg = jnp.clip(xg, 0.0, 100.0) ** (1.0 / pw)
    o = _mmT(xg, wg_ref[...]) + bg_ref[...]
    out_ref[...] = o
    iota = lax.broadcasted_iota(jnp.int32, (N // GRP, NCLS), 1)
    mx = jnp.max(o, axis=1, keepdims=True)
    ypred_ref[...] = jnp.min(jnp.where(o == mx, iota, NCLS), axis=1,
                             keepdims=True)


_tc_prep = _tc(_prep_body, ((NP, D), _f32), ((NP, 1), _f32), ((NP, 1), _f32))
_tc_mid_relu = _tc(_make_mid_body(_relu), ((NP, D), _f32))
_tc_mid_leaky = _tc(_make_mid_body(_leaky), ((NP, D), _f32))
_tc_merge = _tc(_merge_body, ((NP, D), _f32))
_tc_final = _tc(_final_body, ((N // GRP, NCLS), _f32), ((N // GRP, 1), jnp.int32))


def _prep_edges(ei):
    pad = jnp.full((EPAD - E,), DUMMY, jnp.int32)
    r = jnp.concatenate([ei[0], pad]).reshape(NROWS, CHUNK)
    c = jnp.concatenate([ei[1], pad]).reshape(NROWS, CHUNK)
    return r, c


def kernel(x, internal_edge_index, all_edge_index, params):
    p = params
    x_pad = jnp.pad(x, ((0, NP - N), (0, 0)))
    ri, ci = _prep_edges(internal_edge_index)
    ra, ca = _prep_edges(all_edge_index)
    zeros_d = jnp.zeros((RPT, D), _f32)
    ones_d = jnp.ones((CHUNK, D), _f32)
    r2 = lambda v: v.reshape(1, -1)

    degp = _sc_deg(ci, ca, ones_d, zeros_d)[:, :, :, 0:1]
    xs, dis_i, dis_a = _tc_prep(x_pad, degp, p['W_it'], r2(p['b_it']),
                                p['iW1'], r2(p['ib1']))
    yp = _sc_agg(xs, ri, ci, zeros_d)
    xs = _tc_mid_relu(yp, xs, dis_i, r2(p['ig1']), r2(p['ibt1']),
                      p['iW2'], r2(p['ib2']), dis_i)[0]
    yp = _sc_agg(xs, ri, ci, zeros_d)
    xs = _tc_mid_relu(yp, xs, dis_i, r2(p['ig2']), r2(p['ibt2']),
                      p['iW3'], r2(p['ib3']), dis_i)[0]
    yp = _sc_agg(xs, ri, ci, zeros_d)
    xs = _tc_merge(yp, xs, dis_i, r2(p['ig3']), r2(p['ibt3']), x_pad,
                   p['W_m'][:, :D], p['W_m'][:, D:], r2(p['b_m']),
                   p['gW1'], r2(p['gb1']), dis_a)[0]
    yp = _sc_agg(xs, ra, ca, zeros_d)
    xs = _tc_mid_leaky(yp, xs, dis_a, r2(p['gg1']), r2(p['gbt1']),
                       p['gW2'], r2(p['gb2']), dis_a)[0]
    yp = _sc_agg(xs, ra, ca, zeros_d)
    xs = _tc_mid_leaky(yp, xs, dis_a, r2(p['gg2']), r2(p['gbt2']),
                       p['gW3'], r2(p['gb3']), dis_a)[0]
    yp = _sc_agg(xs, ra, ca, zeros_d)
    out, ypred = _tc_final(yp, xs, dis_a, r2(p['gg3']), r2(p['gbt3']),
                           p['p_pow'].reshape(1, 1), p['Wg'], r2(p['bg']))
    return out, ypred.reshape(-1)
